# baseline (speedup 1.0000x reference)
import itertools

import jax
import jax.numpy as jnp
import numpy as np
from jax.experimental import pallas as pl
from jax.experimental.pallas import tpu as pltpu

_N, _C, _S, _D = 16, 50, 4, 256
_G = 3 * _D
_ROWS = _N * _S
_TILE = 320


def _build_tables():
    a_rows, k_sizes = [], []
    for k in range(1, _S + 1):
        for subset in itertools.combinations(range(_S), k):
            for p in itertools.permutations(subset):
                a_rows.append(list(p) + [0] * (_S - k))
                k_sizes.append(k)
    A = np.array(a_rows, np.int32)
    K = np.array(k_sizes, np.int32)
    ncand = A.shape[0]
    msel = np.zeros((_S * _ROWS, ncand), np.float32)
    rtw = np.zeros((_S * ncand, _ROWS), np.float32)
    indcode = np.zeros((ncand,), np.float32)
    for a in range(ncand):
        for s in range(K[a]):
            j = A[a, s]
            for g in range(_N):
                msel[j * _ROWS + g * _S + s, a] = 1.0
                rtw[j * ncand + a, g * _S + s] = 1.0
        indcode[a] = float(sum(1 << int(s) for s in A[a, : K[a]]))
    kvec = K.astype(np.float32)[None, :]
    return msel, rtw, indcode, kvec, ncand


_MSEL_NP, _RTW_NP, _INDCODE_NP, _KVEC_NP, _NCAND = _build_tables()


def _roll_up(x, k):
    return jnp.concatenate([x[k:], x[:k]], axis=0)


def _roll_down(x, k):
    return jnp.concatenate([x[_ROWS - k:], x[:_ROWS - k]], axis=0)


def _rnn_clusterer_body(xall_ref, wih_ref, whh_ref, bih_ref, bhh_ref,
                        h0_ref, validm_ref, msel_ref, rtw_ref, kvec_ref,
                        out_ref, gis_ref):
    f32 = jnp.float32
    bf16 = jnp.bfloat16
    wih = wih_ref[:]
    bih = bih_ref[:]

    def precompute(t, carry):
        rows = pl.ds(t * _TILE, _TILE)
        gi = jnp.dot(xall_ref[rows, :], wih, preferred_element_type=f32) + bih
        hi = gi.astype(bf16)
        r1 = gi - hi.astype(f32)
        lo = r1.astype(bf16)
        lo2 = (r1 - lo.astype(f32)).astype(bf16)
        for k in range(_TILE // _ROWS):
            base = (t * (_TILE // _ROWS) + k) * 3 * _ROWS
            sl = slice(k * _ROWS, (k + 1) * _ROWS)
            gis_ref[pl.ds(base, _ROWS), :] = hi[sl]
            gis_ref[pl.ds(base + _ROWS, _ROWS), :] = lo[sl]
            gis_ref[pl.ds(base + 2 * _ROWS, _ROWS), :] = lo2[sl]
        return carry

    jax.lax.fori_loop(0, (_C * _ROWS) // _TILE, precompute, 0)

    whh = whh_ref[:]
    bhh = bhh_ref[:]
    msel = [msel_ref[_ROWS * j:_ROWS * j + _ROWS, :] for j in range(_S)]
    kvec = kvec_ref[:]

    io_r = jax.lax.broadcasted_iota(jnp.int32, (_ROWS, _ROWS), 0)
    io_q = jax.lax.broadcasted_iota(jnp.int32, (_ROWS, _ROWS), 1)
    grpmask = ((io_q // _S) == (io_r // _S)).astype(f32)
    io_rcol = jax.lax.broadcasted_iota(jnp.int32, (_ROWS, 1), 0)
    rowmask = [(io_rcol % _S == j).astype(f32) for j in range(_S)]
    repmask = [(io_q == (io_r // _S) * _S + j) for j in range(_S)]
    iota_a = io_q

    hidden0 = jnp.broadcast_to(h0_ref[:], (_ROWS, _D))
    acc0 = jnp.zeros((_ROWS, 2), f32)

    def step(c, carry):
        hidden, acc = carry
        x_c = xall_ref[pl.ds(c * _ROWS, _ROWS), :]
        valid = validm_ref[c]

        gh = jnp.dot(hidden, whh, preferred_element_type=f32) + bhh

        gfull = jax.lax.dot_general(
            x_c, hidden, (((1,), (1,)), ((), ())),
            preferred_element_type=f32)
        logits = jnp.concatenate(
            [jnp.sum(jnp.where(repmask[j], gfull, 0.0), axis=1, keepdims=True)
             for j in range(_S)], axis=1)

        m = jnp.max(logits, axis=1, keepdims=True)
        e = jnp.exp(logits - m)
        prob = e / jnp.sum(e, axis=1, keepdims=True)
        logp = jnp.log(prob)

        perrow = sum(logp[:, j:j + 1] * msel[j] for j in range(_S))
        ssum = ((perrow + _roll_up(perrow, 1)) + _roll_up(perrow, 2)) \
            + _roll_up(perrow, 3)
        cand = -(ssum / kvec)

        masked = jnp.where(valid > jnp.bfloat16(0.5), cand, 1e30)
        best_loss = jnp.min(masked, axis=1, keepdims=True)
        finite = (best_loss < 1e29).astype(f32)
        acc = acc + jnp.concatenate([best_loss * finite, finite], axis=1)

        eq = masked == best_loss
        idx = jnp.min(jnp.where(eq, iota_a, _NCAND), axis=1, keepdims=True)
        idx = rowmask[0] * idx.astype(f32) + sum(
            rowmask[j] * _roll_down(idx.astype(f32), j)
            for j in range(1, _S))
        onehot = (iota_a.astype(f32) == idx).astype(f32) * finite

        lhsp = jnp.concatenate([rowmask[j] * onehot for j in range(_S)],
                               axis=1)
        p = grpmask * jnp.dot(lhsp, rtw_ref[:], preferred_element_type=f32)
        upd = jnp.sum(p, axis=1, keepdims=True)
        pb = p.astype(bf16)
        lhs3 = jnp.concatenate([pb, pb, pb], axis=1)
        gi_sel = jnp.dot(lhs3, gis_ref[pl.ds(c * 3 * _ROWS, 3 * _ROWS), :],
                         preferred_element_type=f32)

        r = jax.nn.sigmoid(gi_sel[:, :_D] + gh[:, :_D])
        z = jax.nn.sigmoid(gi_sel[:, _D:2 * _D] + gh[:, _D:2 * _D])
        n_ = jnp.tanh(gi_sel[:, 2 * _D:] + r * gh[:, 2 * _D:])
        newh = (1.0 - z) * n_ + z * hidden
        hidden = jnp.where(upd > 0.5, newh, hidden)
        return hidden, acc

    def step2(i, carry):
        return step(2 * i + 1, step(2 * i, carry))

    _, acc = jax.lax.fori_loop(0, _C // 2, step2, (hidden0, acc0))
    out_ref[:] = acc[:, 0:1] / (acc[:, 1:2] + 1e-6)


def kernel(spk_emb, label, rnn_init_hidden, W_ih, W_hh, b_ih, b_hh):
    xall = spk_emb.transpose(1, 0, 2, 3).reshape(_C * _ROWS, _D)
    codes = (label.astype(jnp.int32)
             * (1 << jnp.arange(_S, dtype=jnp.int32))).sum(-1)
    codes_rows = jnp.repeat(codes.T, _S, axis=1)
    validm = (codes_rows[:, :, None].astype(jnp.float32)
              == jnp.asarray(_INDCODE_NP)[None, None, :]
              ).astype(jnp.bfloat16)

    out = pl.pallas_call(
        _rnn_clusterer_body,
        out_shape=jax.ShapeDtypeStruct((_ROWS, 1), jnp.float32),
        scratch_shapes=[pltpu.VMEM((_C * _ROWS * 3, _G), jnp.bfloat16)],
    )(
        xall,
        W_ih.T,
        W_hh.T,
        b_ih.reshape(1, _G),
        b_hh.reshape(1, _G),
        rnn_init_hidden,
        validm,
        jnp.asarray(_MSEL_NP),
        jnp.asarray(_RTW_NP),
        jnp.asarray(_KVEC_NP),
    )
    return out.reshape(_N, _S)[:, 0]


# speedup vs baseline: 1.0890x; 1.0890x over previous
import itertools

import jax
import jax.numpy as jnp
import numpy as np
from jax.experimental import pallas as pl
from jax.experimental.pallas import tpu as pltpu

_N, _C, _S, _D = 16, 50, 4, 256
_G = 3 * _D
_ROWS = _N * _S
_TILE = 320


def _build_tables():
    a_rows, k_sizes = [], []
    for k in range(1, _S + 1):
        for subset in itertools.combinations(range(_S), k):
            for p in itertools.permutations(subset):
                a_rows.append(list(p) + [0] * (_S - k))
                k_sizes.append(k)
    A = np.array(a_rows, np.int32)
    K = np.array(k_sizes, np.int32)
    ncand = A.shape[0]
    msel = np.zeros((_S * _ROWS, ncand), np.float32)
    rtw = np.zeros((_S * ncand, _ROWS), np.float32)
    indcode = np.zeros((ncand,), np.float32)
    for a in range(ncand):
        for s in range(K[a]):
            j = A[a, s]
            for g in range(_N):
                msel[j * _ROWS + g * _S + s, a] = 1.0
                rtw[j * ncand + a, g * _S + s] = 1.0
        indcode[a] = float(sum(1 << int(s) for s in A[a, : K[a]]))
    kvec = K.astype(np.float32)[None, :]
    return msel, rtw, indcode, kvec, ncand


_MSEL_NP, _RTW_NP, _INDCODE_NP, _KVEC_NP, _NCAND = _build_tables()


def _roll_up(x, k):
    return jnp.concatenate([x[k:], x[:k]], axis=0)


def _roll_down(x, k):
    return jnp.concatenate([x[_ROWS - k:], x[:_ROWS - k]], axis=0)


def _rnn_clusterer_body(xall_ref, wih_ref, whh_ref, bih_ref, bhh_ref,
                        h0_ref, validm_ref, msel_ref, rtw_ref, kvec_ref,
                        out_ref, gis_ref):
    f32 = jnp.float32
    bf16 = jnp.bfloat16
    wih = wih_ref[:]
    bih = bih_ref[:]

    def precompute(t, carry):
        rows = pl.ds(t * _TILE, _TILE)
        gi = jnp.dot(xall_ref[rows, :], wih, preferred_element_type=f32) + bih
        hi = gi.astype(bf16)
        r1 = gi - hi.astype(f32)
        lo = r1.astype(bf16)
        lo2 = (r1 - lo.astype(f32)).astype(bf16)
        for k in range(_TILE // _ROWS):
            base = (t * (_TILE // _ROWS) + k) * 3 * _ROWS
            sl = slice(k * _ROWS, (k + 1) * _ROWS)
            gis_ref[pl.ds(base, _ROWS), :] = hi[sl]
            gis_ref[pl.ds(base + _ROWS, _ROWS), :] = lo[sl]
            gis_ref[pl.ds(base + 2 * _ROWS, _ROWS), :] = lo2[sl]
        return carry

    jax.lax.fori_loop(0, (_C * _ROWS) // _TILE, precompute, 0)

    whh = whh_ref[:]
    bhh = bhh_ref[:]
    msel = [msel_ref[_ROWS * j:_ROWS * j + _ROWS, :] for j in range(_S)]
    kvec = kvec_ref[:]

    io_r = jax.lax.broadcasted_iota(jnp.int32, (_ROWS, _ROWS), 0)
    io_q = jax.lax.broadcasted_iota(jnp.int32, (_ROWS, _ROWS), 1)
    grpmask = ((io_q // _S) == (io_r // _S)).astype(f32)
    io_rcol = jax.lax.broadcasted_iota(jnp.int32, (_ROWS, 1), 0)
    rowmask = [(io_rcol % _S == j).astype(f32) for j in range(_S)]
    repmask = [(io_q == (io_r // _S) * _S + j) for j in range(_S)]
    iota_a = io_q

    hidden0 = jnp.broadcast_to(h0_ref[:], (_ROWS, _D))
    acc0 = jnp.zeros((_ROWS, 2), f32)

    def step(c, carry):
        hidden, acc = carry
        x_c = xall_ref[pl.ds(c * _ROWS, _ROWS), :]
        valid = validm_ref[c]

        gh = jnp.dot(hidden, whh, preferred_element_type=f32) + bhh

        gfull = jax.lax.dot_general(
            x_c, hidden, (((1,), (1,)), ((), ())),
            preferred_element_type=f32)
        logits = jnp.concatenate(
            [jnp.sum(jnp.where(repmask[j], gfull, 0.0), axis=1, keepdims=True)
             for j in range(_S)], axis=1)

        m = jnp.max(logits, axis=1, keepdims=True)
        e = jnp.exp(logits - m)
        prob = e / jnp.sum(e, axis=1, keepdims=True)
        logp = jnp.log(prob)

        perrow = sum(logp[:, j:j + 1] * msel[j] for j in range(_S))
        ssum = ((perrow + _roll_up(perrow, 1)) + _roll_up(perrow, 2)) \
            + _roll_up(perrow, 3)
        cand = -(ssum / kvec)

        masked = jnp.where(valid > jnp.bfloat16(0.5), cand, 1e30)
        best_loss = jnp.min(masked, axis=1, keepdims=True)
        finite = (best_loss < 1e29).astype(f32)
        acc = acc + jnp.concatenate([best_loss * finite, finite], axis=1)

        idx = jnp.argmin(masked, axis=1, keepdims=True)
        idx = rowmask[0] * idx.astype(f32) + sum(
            rowmask[j] * _roll_down(idx.astype(f32), j)
            for j in range(1, _S))
        onehot = (iota_a.astype(f32) == idx).astype(f32) * finite

        lhsp = jnp.concatenate([rowmask[j] * onehot for j in range(_S)],
                               axis=1)
        p = grpmask * jnp.dot(lhsp, rtw_ref[:], preferred_element_type=f32)
        upd = jnp.sum(p, axis=1, keepdims=True)
        pb = p.astype(bf16)
        lhs3 = jnp.concatenate([pb, pb, pb], axis=1)
        gi_sel = jnp.dot(lhs3, gis_ref[pl.ds(c * 3 * _ROWS, 3 * _ROWS), :],
                         preferred_element_type=f32)

        r = jax.nn.sigmoid(gi_sel[:, :_D] + gh[:, :_D])
        z = jax.nn.sigmoid(gi_sel[:, _D:2 * _D] + gh[:, _D:2 * _D])
        n_ = jnp.tanh(gi_sel[:, 2 * _D:] + r * gh[:, 2 * _D:])
        newh = (1.0 - z) * n_ + z * hidden
        hidden = jnp.where(upd > 0.5, newh, hidden)
        return hidden, acc

    def step2(i, carry):
        return step(2 * i + 1, step(2 * i, carry))

    _, acc = jax.lax.fori_loop(0, _C // 2, step2, (hidden0, acc0))
    out_ref[:] = acc[:, 0:1] / (acc[:, 1:2] + 1e-6)


def kernel(spk_emb, label, rnn_init_hidden, W_ih, W_hh, b_ih, b_hh):
    xall = spk_emb.transpose(1, 0, 2, 3).reshape(_C * _ROWS, _D)
    codes = (label.astype(jnp.int32)
             * (1 << jnp.arange(_S, dtype=jnp.int32))).sum(-1)
    codes_rows = jnp.repeat(codes.T, _S, axis=1)
    validm = (codes_rows[:, :, None].astype(jnp.float32)
              == jnp.asarray(_INDCODE_NP)[None, None, :]
              ).astype(jnp.bfloat16)

    out = pl.pallas_call(
        _rnn_clusterer_body,
        out_shape=jax.ShapeDtypeStruct((_ROWS, 1), jnp.float32),
        scratch_shapes=[pltpu.VMEM((_C * _ROWS * 3, _G), jnp.bfloat16)],
    )(
        xall,
        W_ih.T,
        W_hh.T,
        b_ih.reshape(1, _G),
        b_hh.reshape(1, _G),
        rnn_init_hidden,
        validm,
        jnp.asarray(_MSEL_NP),
        jnp.asarray(_RTW_NP),
        jnp.asarray(_KVEC_NP),
    )
    return out.reshape(_N, _S)[:, 0]
